# Initial kernel scaffold; baseline (speedup 1.0000x reference)
import jax
import jax.numpy as jnp
from jax.experimental import pallas as pl


def kernel(x, x_embed_weight, pos_embed):
    raise NotImplementedError("write your pallas kernel here")


# speedup vs baseline: 3.9905x; 3.9905x over previous
import functools

import jax
import jax.numpy as jnp
from jax.experimental import pallas as pl
from jax.experimental.pallas import tpu as pltpu

N_EMBED_VALS = 256
N_PIXELS = 196
N_STATES = 256
BATCH = 4096

BB = 32


def _embed_kernel(x_ref, tab_ref, pos_ref, out_ref):
    tab = tab_ref[...]
    pos = pos_ref[...]
    iota_v = jax.lax.broadcasted_iota(jnp.int32, (N_EMBED_VALS, N_PIXELS), 0)
    for b in range(BB):
        idx = x_ref[b, :]
        onehot = (iota_v == idx[None, :]).astype(jnp.bfloat16)
        out_ref[b] = (
            jax.lax.dot(tab, onehot, preferred_element_type=jnp.float32) + pos
        )


@jax.jit
def kernel(x, x_embed_weight, pos_embed):
    tab_t = x_embed_weight.T.astype(jnp.bfloat16)
    pos_t = pos_embed.T
    grid = (BATCH // BB,)
    return pl.pallas_call(
        _embed_kernel,
        grid=grid,
        in_specs=[
            pl.BlockSpec((BB, N_PIXELS), lambda i: (i, 0)),
            pl.BlockSpec((N_STATES, N_EMBED_VALS), lambda i: (0, 0)),
            pl.BlockSpec((N_STATES, N_PIXELS), lambda i: (0, 0)),
        ],
        out_specs=pl.BlockSpec((BB, N_STATES, N_PIXELS), lambda i: (i, 0, 0)),
        out_shape=jax.ShapeDtypeStruct((BATCH, N_STATES, N_PIXELS), jnp.float32),
    )(x, tab_t, pos_t)
